# Initial kernel scaffold; baseline (speedup 1.0000x reference)
import jax
import jax.numpy as jnp
from jax.experimental import pallas as pl


def kernel(node_features, mesh_edge_features, mesh_senders, mesh_receivers, world_edge_features, world_senders, world_receivers, params):
    raise NotImplementedError("write your pallas kernel here")


# speedup vs baseline: 2.7325x; 2.7325x over previous
import functools

import jax
import jax.numpy as jnp
from jax import lax
from jax.experimental import pallas as pl
from jax.experimental.pallas import tpu as pltpu
from jax.experimental.pallas import tpu_sc as plsc

_N = 10000
_D = 128
_E_MESH = 320000
_E_WORLD = 160000
_NW = 32
_CH = 256


def _precompute_tables(nf, w_stack):
    blk = 1000
    grid = _N // blk

    def body(nf_ref, w_ref, oa_m, ob_m, oa_w, ob_w):
        x = nf_ref[...]
        oa_m[...] = jnp.dot(x, w_ref[0], preferred_element_type=jnp.float32)
        ob_m[...] = jnp.dot(x, w_ref[1], preferred_element_type=jnp.float32)
        oa_w[...] = jnp.dot(x, w_ref[2], preferred_element_type=jnp.float32)
        ob_w[...] = jnp.dot(x, w_ref[3], preferred_element_type=jnp.float32)

    out = pl.pallas_call(
        body,
        grid=(grid,),
        in_specs=[
            pl.BlockSpec((blk, _D), lambda i: (i, 0)),
            pl.BlockSpec((4, _D, _D), lambda i: (0, 0, 0)),
        ],
        out_specs=[pl.BlockSpec((blk, _D), lambda i: (i, 0))] * 4,
        out_shape=[jax.ShapeDtypeStruct((_N, _D), jnp.float32)] * 4,
    )(nf, w_stack)
    return out


def _sc_gather(tam, tbm, taw, tbw, ms, mr, ws, wr):
    mesh = plsc.VectorSubcoreMesh(core_axis_name="c", subcore_axis_name="s")

    @functools.partial(
        pl.kernel,
        out_type=[jax.ShapeDtypeStruct((_E_MESH, _D), jnp.float32)] * 2
        + [jax.ShapeDtypeStruct((_E_WORLD, _D), jnp.float32)] * 2,
        mesh=mesh,
        scratch_types=[
            pltpu.VMEM((2, 128), jnp.int32),
            pltpu.VMEM((_CH, _D), jnp.float32),
            pltpu.SemaphoreType.DMA,
        ],
    )
    def k(tam_r, tbm_r, taw_r, tbw_r, ms_r, mr_r, ws_r, wr_r,
          gam, gbm, gaw, gbw, idx_v, rows_v, sem):
        wid = lax.axis_index("c") * 16 + lax.axis_index("s")
        jobs = [
            (tam_r, ms_r, gam, _E_MESH),
            (tbm_r, mr_r, gbm, _E_MESH),
            (taw_r, ws_r, gaw, _E_WORLD),
            (tbw_r, wr_r, gbw, _E_WORLD),
        ]
        for tab, idx, out, n_edges in jobs:
            nchunks = n_edges // _CH
            niter = (nchunks + _NW - 1) // _NW

            def body(step, carry, tab=tab, idx=idx, out=out, nchunks=nchunks):
                cid = wid + step * _NW

                @pl.when(cid < nchunks)
                def _():
                    base = cid * _CH
                    pltpu.sync_copy(idx.at[pl.ds(base, 128)], idx_v.at[0])
                    pltpu.sync_copy(idx.at[pl.ds(base + 128, 128)], idx_v.at[1])
                    c0 = pltpu.async_copy(tab.at[idx_v.at[0]],
                                          rows_v.at[pl.ds(0, 128)], sem)
                    c1 = pltpu.async_copy(tab.at[idx_v.at[1]],
                                          rows_v.at[pl.ds(128, 128)], sem)
                    c0.wait()
                    c1.wait()
                    pltpu.sync_copy(rows_v, out.at[pl.ds(base, _CH)])

                return carry

            lax.fori_loop(0, niter, body, None)

    return k(tam, tbm, taw, tbw, ms, mr, ws, wr)


def _edge_mlp(ga, gb, ef, w1e, b1, w2, b2, ln_g, ln_b):
    n_edges = ga.shape[0]
    blk = 640
    grid = n_edges // blk

    def body(ga_r, gb_r, ef_r, w1_r, b1_r, w2_r, b2_r, g_r, be_r,
             new_r, out_r):
        ef = ef_r[...]
        pre = (ga_r[...] + gb_r[...]
               + jnp.dot(ef, w1_r[...], preferred_element_type=jnp.float32)
               + b1_r[...])
        h = jnp.maximum(pre, 0.0)
        o = jnp.dot(h, w2_r[...], preferred_element_type=jnp.float32) + b2_r[...]
        mu = jnp.mean(o, axis=-1, keepdims=True)
        var = jnp.mean((o - mu) ** 2, axis=-1, keepdims=True)
        ln = (o - mu) * lax.rsqrt(var + 1e-5) * g_r[...] + be_r[...]
        new_r[...] = ln
        out_r[...] = ln + ef

    row = pl.BlockSpec((blk, _D), lambda i: (i, 0))
    full = pl.BlockSpec((_D, _D), lambda i: (0, 0))
    vec = pl.BlockSpec((1, _D), lambda i: (0, 0))
    return pl.pallas_call(
        body,
        grid=(grid,),
        in_specs=[row, row, row, full, vec, full, vec, vec, vec],
        out_specs=[row, row],
        out_shape=[jax.ShapeDtypeStruct((n_edges, _D), jnp.float32)] * 2,
    )(ga, gb, ef, w1e, b1.reshape(1, _D), w2, b2.reshape(1, _D),
      ln_g.reshape(1, _D), ln_b.reshape(1, _D))


def _sc_scatter(new_m, mr, new_w, wr):
    mesh = plsc.VectorSubcoreMesh(core_axis_name="c", subcore_axis_name="s")
    zrows = 80
    nzchunks = _N // zrows

    @functools.partial(
        pl.kernel,
        out_type=[jax.ShapeDtypeStruct((2 * _N, _D), jnp.float32)] * 2,
        mesh=mesh,
        scratch_types=[
            pltpu.VMEM((zrows, _D), jnp.float32),
            pltpu.VMEM((2, 128), jnp.int32),
            pltpu.VMEM((_CH, _D), jnp.float32),
            pltpu.VMEM_SHARED((_N, _D), jnp.float32),
            pltpu.SemaphoreType.DMA,
        ],
    )
    def k(nm_r, mr_r, nw_r, wr_r, aggm, aggw, zbuf, idx_v, rows_v, acc, sem):
        core = lax.axis_index("c")
        sub = lax.axis_index("s")
        wid = core * 16 + sub

        zero16 = jnp.zeros((16,), jnp.float32)

        def zbody(i, carry):
            for j in range(_D // 16):
                zbuf[i, pl.ds(j * 16, 16)] = zero16
            return carry

        lax.fori_loop(0, zrows, zbody, None)

        def zero_acc(step, carry):
            cid = sub + step * 16

            @pl.when(cid < nzchunks)
            def _():
                pltpu.sync_copy(zbuf, acc.at[pl.ds(cid * zrows, zrows)])

            return carry

        jobs = [(nm_r, mr_r, aggm, _E_MESH), (nw_r, wr_r, aggw, _E_WORLD)]
        for src, idx, out, n_edges in jobs:
            lax.fori_loop(0, (nzchunks + 15) // 16, zero_acc, None)
            plsc.subcore_barrier()

            nchunks = n_edges // _CH
            niter = (nchunks + _NW - 1) // _NW

            def body(step, carry, src=src, idx=idx, nchunks=nchunks):
                cid = wid + step * _NW

                @pl.when(cid < nchunks)
                def _():
                    base = cid * _CH
                    pltpu.sync_copy(idx.at[pl.ds(base, 128)], idx_v.at[0])
                    pltpu.sync_copy(idx.at[pl.ds(base + 128, 128)], idx_v.at[1])
                    pltpu.sync_copy(src.at[pl.ds(base, _CH)], rows_v)
                    pltpu.sync_copy(rows_v.at[pl.ds(0, 128)],
                                    acc.at[idx_v.at[0]], add=True)
                    pltpu.sync_copy(rows_v.at[pl.ds(128, 128)],
                                    acc.at[idx_v.at[1]], add=True)

                return carry

            lax.fori_loop(0, niter, body, None)
            plsc.subcore_barrier()

            def writeout(step, carry, out=out):
                cid = sub + step * 16

                @pl.when(cid < nzchunks)
                def _():
                    pltpu.sync_copy(
                        acc.at[pl.ds(cid * zrows, zrows)],
                        out.at[pl.ds(core * _N + cid * zrows, zrows)])

                return carry

            lax.fori_loop(0, (nzchunks + 15) // 16, writeout, None)
            plsc.subcore_barrier()

    return k(new_m, mr, new_w, wr)


def _node_mlp(nf, am0, am1, aw0, aw1, w_stack, b1, w2, b2, ln_g, ln_b):
    blk = 1000
    grid = _N // blk

    def body(nf_r, am0_r, am1_r, aw0_r, aw1_r, w_r, b1_r, w2_r, b2_r,
             g_r, be_r, out_r):
        nfx = nf_r[...]
        agg_m = am0_r[...] + am1_r[...]
        agg_w = aw0_r[...] + aw1_r[...]
        pre = (jnp.dot(nfx, w_r[0], preferred_element_type=jnp.float32)
               + jnp.dot(agg_m, w_r[1], preferred_element_type=jnp.float32)
               + jnp.dot(agg_w, w_r[2], preferred_element_type=jnp.float32)
               + b1_r[...])
        h = jnp.maximum(pre, 0.0)
        o = jnp.dot(h, w2_r[...], preferred_element_type=jnp.float32) + b2_r[...]
        mu = jnp.mean(o, axis=-1, keepdims=True)
        var = jnp.mean((o - mu) ** 2, axis=-1, keepdims=True)
        ln = (o - mu) * lax.rsqrt(var + 1e-5) * g_r[...] + be_r[...]
        out_r[...] = ln + nfx

    row = pl.BlockSpec((blk, _D), lambda i: (i, 0))
    full = pl.BlockSpec((_D, _D), lambda i: (0, 0))
    vec = pl.BlockSpec((1, _D), lambda i: (0, 0))
    return pl.pallas_call(
        body,
        grid=(grid,),
        in_specs=[row, row, row, row, row,
                  pl.BlockSpec((3, _D, _D), lambda i: (0, 0, 0)),
                  vec, full, vec, vec, vec],
        out_specs=row,
        out_shape=jax.ShapeDtypeStruct((_N, _D), jnp.float32),
    )(nf, am0, am1, aw0, aw1, w_stack, b1.reshape(1, _D), w2,
      b2.reshape(1, _D), ln_g.reshape(1, _D), ln_b.reshape(1, _D))


def kernel(node_features, mesh_edge_features, mesh_senders, mesh_receivers,
           world_edge_features, world_senders, world_receivers, params):
    pm, pw, pn = params["mesh_edge"], params["world_edge"], params["node"]

    w_gather = jnp.stack([pm["W1"][:_D], pm["W1"][_D:2 * _D],
                          pw["W1"][:_D], pw["W1"][_D:2 * _D]])
    tam, tbm, taw, tbw = _precompute_tables(node_features, w_gather)

    ms = mesh_senders.astype(jnp.int32)
    mr = mesh_receivers.astype(jnp.int32)
    ws = world_senders.astype(jnp.int32)
    wr = world_receivers.astype(jnp.int32)

    gam, gbm, gaw, gbw = _sc_gather(tam, tbm, taw, tbw, ms, mr, ws, wr)

    new_m, out_m = _edge_mlp(gam, gbm, mesh_edge_features,
                             pm["W1"][2 * _D:], pm["b1"], pm["W2"], pm["b2"],
                             pm["ln_g"], pm["ln_b"])
    new_w, out_w = _edge_mlp(gaw, gbw, world_edge_features,
                             pw["W1"][2 * _D:], pw["b1"], pw["W2"], pw["b2"],
                             pw["ln_g"], pw["ln_b"])

    aggm2, aggw2 = _sc_scatter(new_m, mr, new_w, wr)

    w_node = jnp.stack([pn["W1"][:_D], pn["W1"][_D:2 * _D], pn["W1"][2 * _D:]])
    new_nodes = _node_mlp(node_features,
                          aggm2[:_N], aggm2[_N:], aggw2[:_N], aggw2[_N:],
                          w_node, pn["b1"], pn["W2"], pn["b2"],
                          pn["ln_g"], pn["ln_b"])
    return new_nodes, out_m, out_w


# speedup vs baseline: 3.3134x; 1.2126x over previous
import functools

import jax
import jax.numpy as jnp
from jax import lax
from jax.experimental import pallas as pl
from jax.experimental.pallas import tpu as pltpu
from jax.experimental.pallas import tpu_sc as plsc

_N = 10000
_D = 128
_E_MESH = 320000
_E_WORLD = 160000
_NW = 32
_CH = 256


def _precompute_tables(nf, w_stack):
    blk = 1000
    grid = _N // blk

    def body(nf_ref, w_ref, oa_m, ob_m, oa_w, ob_w):
        x = nf_ref[...]
        oa_m[...] = jnp.dot(x, w_ref[0], preferred_element_type=jnp.float32)
        ob_m[...] = jnp.dot(x, w_ref[1], preferred_element_type=jnp.float32)
        oa_w[...] = jnp.dot(x, w_ref[2], preferred_element_type=jnp.float32)
        ob_w[...] = jnp.dot(x, w_ref[3], preferred_element_type=jnp.float32)

    out = pl.pallas_call(
        body,
        grid=(grid,),
        in_specs=[
            pl.BlockSpec((blk, _D), lambda i: (i, 0)),
            pl.BlockSpec((4, _D, _D), lambda i: (0, 0, 0)),
        ],
        out_specs=[pl.BlockSpec((blk, _D), lambda i: (i, 0))] * 4,
        out_shape=[jax.ShapeDtypeStruct((_N, _D), jnp.float32)] * 4,
    )(nf, w_stack)
    return out


def _sc_gather(ta, tb, sidx, ridx, n_edges):
    mesh = plsc.VectorSubcoreMesh(core_axis_name="c", subcore_axis_name="s")

    @functools.partial(
        pl.kernel,
        out_type=[jax.ShapeDtypeStruct((n_edges, _D), jnp.float32)] * 2,
        mesh=mesh,
        scratch_types=[
            pltpu.VMEM((2, 128), jnp.int32),
            pltpu.VMEM((_CH, _D), jnp.float32),
            pltpu.SemaphoreType.DMA,
        ],
    )
    def k(ta_r, tb_r, si_r, ri_r, ga, gb, idx_v, rows_v, sem):
        wid = lax.axis_index("c") * 16 + lax.axis_index("s")
        nchunks = n_edges // _CH
        niter = (nchunks + _NW - 1) // _NW
        for tab, idx, out in ((ta_r, si_r, ga), (tb_r, ri_r, gb)):

            def body(step, carry, tab=tab, idx=idx, out=out):
                cid = wid + step * _NW

                @pl.when(cid < nchunks)
                def _():
                    base = cid * _CH
                    pltpu.sync_copy(idx.at[pl.ds(base, 128)], idx_v.at[0])
                    pltpu.sync_copy(idx.at[pl.ds(base + 128, 128)], idx_v.at[1])
                    c0 = pltpu.async_copy(tab.at[idx_v.at[0]],
                                          rows_v.at[pl.ds(0, 128)], sem)
                    c1 = pltpu.async_copy(tab.at[idx_v.at[1]],
                                          rows_v.at[pl.ds(128, 128)], sem)
                    c0.wait()
                    c1.wait()
                    pltpu.sync_copy(rows_v, out.at[pl.ds(base, _CH)])

                return carry

            lax.fori_loop(0, niter, body, None)

    return k(ta, tb, sidx, ridx)


def _edge_mlp(ga, gb, ef, w1e, b1, w2, b2, ln_g, ln_b):
    n_edges = ga.shape[0]
    blk = 640
    grid = n_edges // blk

    def body(ga_r, gb_r, ef_r, w1_r, b1_r, w2_r, b2_r, g_r, be_r,
             new_r, out_r):
        ef = ef_r[...]
        pre = (ga_r[...] + gb_r[...]
               + jnp.dot(ef, w1_r[...], preferred_element_type=jnp.float32)
               + b1_r[...])
        h = jnp.maximum(pre, 0.0)
        o = jnp.dot(h, w2_r[...], preferred_element_type=jnp.float32) + b2_r[...]
        mu = jnp.mean(o, axis=-1, keepdims=True)
        var = jnp.mean((o - mu) ** 2, axis=-1, keepdims=True)
        ln = (o - mu) * lax.rsqrt(var + 1e-5) * g_r[...] + be_r[...]
        new_r[...] = ln
        out_r[...] = ln + ef

    row = pl.BlockSpec((blk, _D), lambda i: (i, 0))
    full = pl.BlockSpec((_D, _D), lambda i: (0, 0))
    vec = pl.BlockSpec((1, _D), lambda i: (0, 0))
    return pl.pallas_call(
        body,
        grid=(grid,),
        in_specs=[row, row, row, full, vec, full, vec, vec, vec],
        out_specs=[row, row],
        out_shape=[jax.ShapeDtypeStruct((n_edges, _D), jnp.float32)] * 2,
    )(ga, gb, ef, w1e, b1.reshape(1, _D), w2, b2.reshape(1, _D),
      ln_g.reshape(1, _D), ln_b.reshape(1, _D))


def _sc_scatter(new_e, ridx, n_edges):
    mesh = plsc.VectorSubcoreMesh(core_axis_name="c", subcore_axis_name="s")
    zrows = 80
    nzchunks = _N // zrows

    @functools.partial(
        pl.kernel,
        out_type=jax.ShapeDtypeStruct((2 * _N, _D), jnp.float32),
        mesh=mesh,
        scratch_types=[
            pltpu.VMEM((zrows, _D), jnp.float32),
            pltpu.VMEM((2, 128), jnp.int32),
            pltpu.VMEM((_CH, _D), jnp.float32),
            pltpu.VMEM_SHARED((_N, _D), jnp.float32),
            pltpu.SemaphoreType.DMA,
        ],
    )
    def k(src, idx, out, zbuf, idx_v, rows_v, acc, sem):
        core = lax.axis_index("c")
        sub = lax.axis_index("s")
        wid = core * 16 + sub

        zero16 = jnp.zeros((16,), jnp.float32)

        def zbody(i, carry):
            for j in range(_D // 16):
                zbuf[i, pl.ds(j * 16, 16)] = zero16
            return carry

        lax.fori_loop(0, zrows, zbody, None)

        def zero_acc(step, carry):
            cid = sub + step * 16

            @pl.when(cid < nzchunks)
            def _():
                pltpu.sync_copy(zbuf, acc.at[pl.ds(cid * zrows, zrows)])

            return carry

        lax.fori_loop(0, (nzchunks + 15) // 16, zero_acc, None)
        plsc.subcore_barrier()

        nchunks = n_edges // _CH
        niter = (nchunks + _NW - 1) // _NW

        def body(step, carry):
            cid = wid + step * _NW

            @pl.when(cid < nchunks)
            def _():
                base = cid * _CH
                pltpu.sync_copy(idx.at[pl.ds(base, 128)], idx_v.at[0])
                pltpu.sync_copy(idx.at[pl.ds(base + 128, 128)], idx_v.at[1])
                pltpu.sync_copy(src.at[pl.ds(base, _CH)], rows_v)
                pltpu.sync_copy(rows_v.at[pl.ds(0, 128)],
                                acc.at[idx_v.at[0]], add=True)
                pltpu.sync_copy(rows_v.at[pl.ds(128, 128)],
                                acc.at[idx_v.at[1]], add=True)

            return carry

        lax.fori_loop(0, niter, body, None)
        plsc.subcore_barrier()

        def writeout(step, carry):
            cid = sub + step * 16

            @pl.when(cid < nzchunks)
            def _():
                pltpu.sync_copy(
                    acc.at[pl.ds(cid * zrows, zrows)],
                    out.at[pl.ds(core * _N + cid * zrows, zrows)])

            return carry

        lax.fori_loop(0, (nzchunks + 15) // 16, writeout, None)
        plsc.subcore_barrier()

    return k(new_e, ridx)


def _node_mlp(nf, am0, am1, aw0, aw1, w_stack, b1, w2, b2, ln_g, ln_b):
    blk = 1000
    grid = _N // blk

    def body(nf_r, am0_r, am1_r, aw0_r, aw1_r, w_r, b1_r, w2_r, b2_r,
             g_r, be_r, out_r):
        nfx = nf_r[...]
        agg_m = am0_r[...] + am1_r[...]
        agg_w = aw0_r[...] + aw1_r[...]
        pre = (jnp.dot(nfx, w_r[0], preferred_element_type=jnp.float32)
               + jnp.dot(agg_m, w_r[1], preferred_element_type=jnp.float32)
               + jnp.dot(agg_w, w_r[2], preferred_element_type=jnp.float32)
               + b1_r[...])
        h = jnp.maximum(pre, 0.0)
        o = jnp.dot(h, w2_r[...], preferred_element_type=jnp.float32) + b2_r[...]
        mu = jnp.mean(o, axis=-1, keepdims=True)
        var = jnp.mean((o - mu) ** 2, axis=-1, keepdims=True)
        ln = (o - mu) * lax.rsqrt(var + 1e-5) * g_r[...] + be_r[...]
        out_r[...] = ln + nfx

    row = pl.BlockSpec((blk, _D), lambda i: (i, 0))
    full = pl.BlockSpec((_D, _D), lambda i: (0, 0))
    vec = pl.BlockSpec((1, _D), lambda i: (0, 0))
    return pl.pallas_call(
        body,
        grid=(grid,),
        in_specs=[row, row, row, row, row,
                  pl.BlockSpec((3, _D, _D), lambda i: (0, 0, 0)),
                  vec, full, vec, vec, vec],
        out_specs=row,
        out_shape=jax.ShapeDtypeStruct((_N, _D), jnp.float32),
    )(nf, am0, am1, aw0, aw1, w_stack, b1.reshape(1, _D), w2,
      b2.reshape(1, _D), ln_g.reshape(1, _D), ln_b.reshape(1, _D))


def kernel(node_features, mesh_edge_features, mesh_senders, mesh_receivers,
           world_edge_features, world_senders, world_receivers, params):
    pm, pw, pn = params["mesh_edge"], params["world_edge"], params["node"]

    w_gather = jnp.stack([pm["W1"][:_D], pm["W1"][_D:2 * _D],
                          pw["W1"][:_D], pw["W1"][_D:2 * _D]])
    tam, tbm, taw, tbw = _precompute_tables(node_features, w_gather)

    ms = mesh_senders.astype(jnp.int32)
    mr = mesh_receivers.astype(jnp.int32)
    ws = world_senders.astype(jnp.int32)
    wr = world_receivers.astype(jnp.int32)

    gam, gbm = _sc_gather(tam, tbm, ms, mr, _E_MESH)
    gaw, gbw = _sc_gather(taw, tbw, ws, wr, _E_WORLD)

    new_m, out_m = _edge_mlp(gam, gbm, mesh_edge_features,
                             pm["W1"][2 * _D:], pm["b1"], pm["W2"], pm["b2"],
                             pm["ln_g"], pm["ln_b"])
    new_w, out_w = _edge_mlp(gaw, gbw, world_edge_features,
                             pw["W1"][2 * _D:], pw["b1"], pw["W2"], pw["b2"],
                             pw["ln_g"], pw["ln_b"])

    aggm2 = _sc_scatter(new_m, mr, _E_MESH)
    aggw2 = _sc_scatter(new_w, wr, _E_WORLD)

    w_node = jnp.stack([pn["W1"][:_D], pn["W1"][_D:2 * _D], pn["W1"][2 * _D:]])
    new_nodes = _node_mlp(node_features,
                          aggm2[:_N], aggm2[_N:], aggw2[:_N], aggw2[_N:],
                          w_node, pn["b1"], pn["W2"], pn["b2"],
                          pn["ln_g"], pn["ln_b"])
    return new_nodes, out_m, out_w


# speedup vs baseline: 3.5239x; 1.0635x over previous
import functools

import jax
import jax.numpy as jnp
from jax import lax
from jax.experimental import pallas as pl
from jax.experimental.pallas import tpu as pltpu
from jax.experimental.pallas import tpu_sc as plsc

_N = 10000
_D = 128
_E_MESH = 320000
_E_WORLD = 160000
_NW = 32
_CH = 256


def _precompute_tables(nf, w_stack):
    blk = 2000
    grid = _N // blk

    def body(nf_ref, w_ref, oa_m, ob_m, oa_w, ob_w):
        x = nf_ref[...]
        for j, o_ref in enumerate((oa_m, ob_m, oa_w, ob_w)):
            o_ref[...] = jnp.dot(x, w_ref[j],
                                 preferred_element_type=jnp.float32)

    out = pl.pallas_call(
        body,
        grid=(grid,),
        in_specs=[
            pl.BlockSpec((blk, _D), lambda i: (i, 0)),
            pl.BlockSpec((4, _D, _D), lambda i: (0, 0, 0)),
        ],
        out_specs=[pl.BlockSpec((blk, _D), lambda i: (i, 0))] * 4,
        out_shape=[jax.ShapeDtypeStruct((_N, _D), jnp.float32)] * 4,
    )(nf, w_stack)
    return out


def _sc_gather(ta, tb, sidx, ridx, n_edges):
    mesh = plsc.VectorSubcoreMesh(core_axis_name="c", subcore_axis_name="s")

    @functools.partial(
        pl.kernel,
        out_type=[jax.ShapeDtypeStruct((n_edges, _D), jnp.float32)] * 2,
        mesh=mesh,
        scratch_types=[
            pltpu.VMEM((2, 128), jnp.int32),
            pltpu.VMEM((_CH, _D), jnp.float32),
            pltpu.VMEM_SHARED((_N, _D), jnp.float32),
            pltpu.SemaphoreType.DMA,
        ],
    )
    def k(ta_r, tb_r, si_r, ri_r, ga, gb, idx_v, rows_v, stab, sem):
        sub = lax.axis_index("s")
        wid = lax.axis_index("c") * 16 + sub
        nchunks = n_edges // _CH
        niter = (nchunks + _NW - 1) // _NW
        zrows, nzchunks = 80, _N // 80
        for tab, idx, out in ((ta_r, si_r, ga), (tb_r, ri_r, gb)):
            def stage(step, carry, tab=tab):
                cid = sub + step * 16

                @pl.when(cid < nzchunks)
                def _():
                    pltpu.sync_copy(tab.at[pl.ds(cid * zrows, zrows)],
                                    stab.at[pl.ds(cid * zrows, zrows)])

                return carry

            lax.fori_loop(0, (nzchunks + 15) // 16, stage, None)
            plsc.subcore_barrier()

            def body(step, carry, idx=idx, out=out):
                cid = wid + step * _NW

                @pl.when(cid < nchunks)
                def _():
                    base = cid * _CH
                    pltpu.sync_copy(idx.at[pl.ds(base, 128)], idx_v.at[0])
                    pltpu.sync_copy(idx.at[pl.ds(base + 128, 128)], idx_v.at[1])
                    c0 = pltpu.async_copy(stab.at[idx_v.at[0]],
                                          rows_v.at[pl.ds(0, 128)], sem)
                    c1 = pltpu.async_copy(stab.at[idx_v.at[1]],
                                          rows_v.at[pl.ds(128, 128)], sem)
                    c0.wait()
                    c1.wait()
                    pltpu.sync_copy(rows_v, out.at[pl.ds(base, _CH)])

                return carry

            lax.fori_loop(0, niter, body, None)
            plsc.subcore_barrier()

    return k(ta, tb, sidx, ridx)


def _edge_mlp(ga, gb, ef, w1e, b1, w2, b2, ln_g, ln_b):
    n_edges = ga.shape[0]
    blk = 640
    grid = n_edges // blk

    def body(ga_r, gb_r, ef_r, w1_r, b1_r, w2_r, b2_r, g_r, be_r,
             new_r, out_r):
        ef = ef_r[...]
        pre = (ga_r[...] + gb_r[...]
               + jnp.dot(ef, w1_r[...], preferred_element_type=jnp.float32)
               + b1_r[...])
        h = jnp.maximum(pre, 0.0)
        o = jnp.dot(h, w2_r[...], preferred_element_type=jnp.float32) + b2_r[...]
        mu = jnp.mean(o, axis=-1, keepdims=True)
        var = jnp.mean((o - mu) ** 2, axis=-1, keepdims=True)
        ln = (o - mu) * lax.rsqrt(var + 1e-5) * g_r[...] + be_r[...]
        new_r[...] = ln
        out_r[...] = ln + ef

    row = pl.BlockSpec((blk, _D), lambda i: (i, 0))
    full = pl.BlockSpec((_D, _D), lambda i: (0, 0))
    vec = pl.BlockSpec((1, _D), lambda i: (0, 0))
    return pl.pallas_call(
        body,
        grid=(grid,),
        in_specs=[row, row, row, full, vec, full, vec, vec, vec],
        out_specs=[row, row],
        out_shape=[jax.ShapeDtypeStruct((n_edges, _D), jnp.float32)] * 2,
    )(ga, gb, ef, w1e, b1.reshape(1, _D), w2, b2.reshape(1, _D),
      ln_g.reshape(1, _D), ln_b.reshape(1, _D))


def _sc_scatter(new_e, ridx, n_edges):
    mesh = plsc.VectorSubcoreMesh(core_axis_name="c", subcore_axis_name="s")
    zrows = 80
    nzchunks = _N // zrows

    @functools.partial(
        pl.kernel,
        out_type=jax.ShapeDtypeStruct((2 * _N, _D), jnp.float32),
        mesh=mesh,
        scratch_types=[
            pltpu.VMEM((zrows, _D), jnp.float32),
            pltpu.VMEM((2, 128), jnp.int32),
            pltpu.VMEM((_CH, _D), jnp.float32),
            pltpu.VMEM_SHARED((_N, _D), jnp.float32),
            pltpu.SemaphoreType.DMA,
        ],
    )
    def k(src, idx, out, zbuf, idx_v, rows_v, acc, sem):
        core = lax.axis_index("c")
        sub = lax.axis_index("s")
        wid = core * 16 + sub

        zero16 = jnp.zeros((16,), jnp.float32)

        def zbody(i, carry):
            for j in range(_D // 16):
                zbuf[i, pl.ds(j * 16, 16)] = zero16
            return carry

        lax.fori_loop(0, zrows, zbody, None)

        def zero_acc(step, carry):
            cid = sub + step * 16

            @pl.when(cid < nzchunks)
            def _():
                pltpu.sync_copy(zbuf, acc.at[pl.ds(cid * zrows, zrows)])

            return carry

        lax.fori_loop(0, (nzchunks + 15) // 16, zero_acc, None)
        plsc.subcore_barrier()

        nchunks = n_edges // _CH
        niter = (nchunks + _NW - 1) // _NW

        def body(step, carry):
            cid = wid + step * _NW

            @pl.when(cid < nchunks)
            def _():
                base = cid * _CH
                pltpu.sync_copy(idx.at[pl.ds(base, 128)], idx_v.at[0])
                pltpu.sync_copy(idx.at[pl.ds(base + 128, 128)], idx_v.at[1])
                pltpu.sync_copy(src.at[pl.ds(base, _CH)], rows_v)
                pltpu.sync_copy(rows_v.at[pl.ds(0, 128)],
                                acc.at[idx_v.at[0]], add=True)
                pltpu.sync_copy(rows_v.at[pl.ds(128, 128)],
                                acc.at[idx_v.at[1]], add=True)

            return carry

        lax.fori_loop(0, niter, body, None)
        plsc.subcore_barrier()

        def writeout(step, carry):
            cid = sub + step * 16

            @pl.when(cid < nzchunks)
            def _():
                pltpu.sync_copy(
                    acc.at[pl.ds(cid * zrows, zrows)],
                    out.at[pl.ds(core * _N + cid * zrows, zrows)])

            return carry

        lax.fori_loop(0, (nzchunks + 15) // 16, writeout, None)
        plsc.subcore_barrier()

    return k(new_e, ridx)


def _node_mlp(nf, am0, am1, aw0, aw1, w_stack, b1, w2, b2, ln_g, ln_b):
    blk = 1000
    grid = _N // blk

    def body(nf_r, am0_r, am1_r, aw0_r, aw1_r, w_r, b1_r, w2_r, b2_r,
             g_r, be_r, out_r):
        nfx = nf_r[...]
        agg_m = am0_r[...] + am1_r[...]
        agg_w = aw0_r[...] + aw1_r[...]
        pre = (jnp.dot(nfx, w_r[0], preferred_element_type=jnp.float32)
               + jnp.dot(agg_m, w_r[1], preferred_element_type=jnp.float32)
               + jnp.dot(agg_w, w_r[2], preferred_element_type=jnp.float32)
               + b1_r[...])
        h = jnp.maximum(pre, 0.0)
        o = jnp.dot(h, w2_r[...], preferred_element_type=jnp.float32) + b2_r[...]
        mu = jnp.mean(o, axis=-1, keepdims=True)
        var = jnp.mean((o - mu) ** 2, axis=-1, keepdims=True)
        ln = (o - mu) * lax.rsqrt(var + 1e-5) * g_r[...] + be_r[...]
        out_r[...] = ln + nfx

    row = pl.BlockSpec((blk, _D), lambda i: (i, 0))
    full = pl.BlockSpec((_D, _D), lambda i: (0, 0))
    vec = pl.BlockSpec((1, _D), lambda i: (0, 0))
    return pl.pallas_call(
        body,
        grid=(grid,),
        in_specs=[row, row, row, row, row,
                  pl.BlockSpec((3, _D, _D), lambda i: (0, 0, 0)),
                  vec, full, vec, vec, vec],
        out_specs=row,
        out_shape=jax.ShapeDtypeStruct((_N, _D), jnp.float32),
    )(nf, am0, am1, aw0, aw1, w_stack, b1.reshape(1, _D), w2,
      b2.reshape(1, _D), ln_g.reshape(1, _D), ln_b.reshape(1, _D))


def kernel(node_features, mesh_edge_features, mesh_senders, mesh_receivers,
           world_edge_features, world_senders, world_receivers, params):
    pm, pw, pn = params["mesh_edge"], params["world_edge"], params["node"]

    w_gather = jnp.stack([pm["W1"][:_D], pm["W1"][_D:2 * _D],
                          pw["W1"][:_D], pw["W1"][_D:2 * _D]])
    tam, tbm, taw, tbw = _precompute_tables(node_features, w_gather)

    ms = mesh_senders.astype(jnp.int32)
    mr = mesh_receivers.astype(jnp.int32)
    ws = world_senders.astype(jnp.int32)
    wr = world_receivers.astype(jnp.int32)

    gam, gbm = _sc_gather(tam, tbm, ms, mr, _E_MESH)
    gaw, gbw = _sc_gather(taw, tbw, ws, wr, _E_WORLD)

    new_m, out_m = _edge_mlp(gam, gbm, mesh_edge_features,
                             pm["W1"][2 * _D:], pm["b1"], pm["W2"], pm["b2"],
                             pm["ln_g"], pm["ln_b"])
    new_w, out_w = _edge_mlp(gaw, gbw, world_edge_features,
                             pw["W1"][2 * _D:], pw["b1"], pw["W2"], pw["b2"],
                             pw["ln_g"], pw["ln_b"])

    aggm2 = _sc_scatter(new_m, mr, _E_MESH)
    aggw2 = _sc_scatter(new_w, wr, _E_WORLD)

    w_node = jnp.stack([pn["W1"][:_D], pn["W1"][_D:2 * _D], pn["W1"][2 * _D:]])
    new_nodes = _node_mlp(node_features,
                          aggm2[:_N], aggm2[_N:], aggw2[:_N], aggw2[_N:],
                          w_node, pn["b1"], pn["W2"], pn["b2"],
                          pn["ln_g"], pn["ln_b"])
    return new_nodes, out_m, out_w
